# baseline (speedup 1.0000x reference)
import functools

import jax
import jax.numpy as jnp
from jax.experimental import pallas as pl
from jax.experimental.pallas import tpu as pltpu
from jax.experimental.pallas import tpu_sc as plsc

EMBED_DIM = 32
CHUNK = 128
CHUNKS_PER_STEP = 8
IDX_PER_STEP = CHUNK * CHUNKS_PER_STEP
OUT_MINOR = CHUNK * EMBED_DIM
TBLOCK = 8192


def _transpose_table(t_t):
    V = t_t.shape[1]

    def body(in_ref, out_ref):
        s = in_ref[...].T.reshape(TBLOCK // 4, 4, EMBED_DIM)
        out_ref[...] = jnp.concatenate(
            [s[:, k, :] for k in range(4)], axis=1
        )

    return pl.pallas_call(
        body,
        grid=(pl.cdiv(V, TBLOCK),),
        in_specs=[pl.BlockSpec((EMBED_DIM, TBLOCK), lambda i: (0, i))],
        out_specs=pl.BlockSpec((TBLOCK // 4, 128), lambda i: (i, 0)),
        out_shape=jax.ShapeDtypeStruct((V // 4, 128), jnp.float32),
        compiler_params=pltpu.CompilerParams(
            dimension_semantics=("parallel",)
        ),
    )(t_t)


@jax.jit
def kernel(x, table):
    B, L = x.shape
    V = table.shape[0]
    N = B * L
    x_t = jnp.transpose(x)
    table_rm = (
        jnp.transpose(table)
        .reshape(EMBED_DIM, V // 4, 4)
        .transpose(1, 2, 0)
        .reshape(V // 4, 4 * EMBED_DIM)
        .reshape(V, EMBED_DIM)
    )
    mesh = plsc.VectorSubcoreMesh(core_axis_name="c", subcore_axis_name="s")
    steps_per_l = B // IDX_PER_STEP
    rows_per_step = IDX_PER_STEP * EMBED_DIM // OUT_MINOR

    @functools.partial(
        pl.kernel,
        out_type=jax.ShapeDtypeStruct((N, EMBED_DIM), table.dtype),
        mesh=mesh,
        compiler_params=pltpu.CompilerParams(use_tc_tiling_on_sc=False),
        scratch_types=[pltpu.SemaphoreType.DMA],
    )
    def gather_kernel(table_hbm, idx_hbm, out_hbm, sem):
        def body(idx_vmem, out_vmem):
            copies = [
                pltpu.async_copy(
                    table_hbm.at[idx_vmem.at[0, pl.ds(j * CHUNK, CHUNK)]],
                    out_vmem.at[pl.ds(j * CHUNK, CHUNK)],
                    sem,
                )
                for j in range(CHUNKS_PER_STEP)
            ]
            for c in copies:
                c.wait()

        pltpu.emit_pipeline(
            body,
            grid=(L, steps_per_l),
            in_specs=[
                pl.BlockSpec((1, IDX_PER_STEP), lambda l, i: (l, i))
            ],
            out_specs=[
                pl.BlockSpec(
                    (IDX_PER_STEP, EMBED_DIM),
                    lambda l, i: (l * steps_per_l + i, 0),
                )
            ],
            core_axis_name=("c", "s"),
            dimension_semantics=(pltpu.PARALLEL, pltpu.PARALLEL),
        )(idx_hbm, out_hbm)

    g = gather_kernel(table_rm, x_t)
    return g.reshape(L, B, EMBED_DIM).transpose(1, 0, 2)


# speedup vs baseline: 1.0603x; 1.0603x over previous
import functools

import jax
import jax.numpy as jnp
from jax.experimental import pallas as pl
from jax.experimental.pallas import tpu as pltpu
from jax.experimental.pallas import tpu_sc as plsc

EMBED_DIM = 32
CHUNK = 128
CHUNKS_PER_STEP = 8
IDX_PER_STEP = CHUNK * CHUNKS_PER_STEP
OUT_MINOR = CHUNK * EMBED_DIM
TBLOCK = 8192


def _transpose_table(t_t):
    V = t_t.shape[1]

    def body(in_ref, out_ref):
        s = in_ref[...].T.reshape(TBLOCK // 4, 4, EMBED_DIM)
        out_ref[...] = jnp.concatenate(
            [s[:, k, :] for k in range(4)], axis=1
        )

    return pl.pallas_call(
        body,
        grid=(pl.cdiv(V, TBLOCK),),
        in_specs=[pl.BlockSpec((EMBED_DIM, TBLOCK), lambda i: (0, i))],
        out_specs=pl.BlockSpec((TBLOCK // 4, 128), lambda i: (i, 0)),
        out_shape=jax.ShapeDtypeStruct((V // 4, 128), jnp.float32),
        compiler_params=pltpu.CompilerParams(
            dimension_semantics=("parallel",)
        ),
    )(t_t)


@jax.jit
def kernel(x, table):
    B, L = x.shape
    V = table.shape[0]
    N = B * L
    x_t = jnp.transpose(x)
    table_c = (
        jnp.transpose(table)
        .reshape(EMBED_DIM, V // 4, 4)
        .transpose(1, 2, 0)
        .reshape(V // 4, 4 * EMBED_DIM)
    )
    table_rm = jax.lax.optimization_barrier(table_c).reshape(V, EMBED_DIM)
    mesh = plsc.VectorSubcoreMesh(core_axis_name="c", subcore_axis_name="s")
    steps_per_l = B // IDX_PER_STEP
    rows_per_step = IDX_PER_STEP * EMBED_DIM // OUT_MINOR

    @functools.partial(
        pl.kernel,
        out_type=jax.ShapeDtypeStruct((N, EMBED_DIM), table.dtype),
        mesh=mesh,
        compiler_params=pltpu.CompilerParams(use_tc_tiling_on_sc=False),
        scratch_types=[pltpu.SemaphoreType.DMA],
    )
    def gather_kernel(table_hbm, idx_hbm, out_hbm, sem):
        def body(idx_vmem, out_vmem):
            copies = [
                pltpu.async_copy(
                    table_hbm.at[idx_vmem.at[0, pl.ds(j * CHUNK, CHUNK)]],
                    out_vmem.at[pl.ds(j * CHUNK, CHUNK)],
                    sem,
                )
                for j in range(CHUNKS_PER_STEP)
            ]
            for c in copies:
                c.wait()

        pltpu.emit_pipeline(
            body,
            grid=(L, steps_per_l),
            in_specs=[
                pl.BlockSpec((1, IDX_PER_STEP), lambda l, i: (l, i))
            ],
            out_specs=[
                pl.BlockSpec(
                    (IDX_PER_STEP, EMBED_DIM),
                    lambda l, i: (l * steps_per_l + i, 0),
                )
            ],
            core_axis_name=("c", "s"),
            dimension_semantics=(pltpu.PARALLEL, pltpu.PARALLEL),
        )(idx_hbm, out_hbm)

    g = gather_kernel(table_rm, x_t)
    return g.reshape(L, B, EMBED_DIM).transpose(1, 0, 2)


# speedup vs baseline: 1.1125x; 1.0492x over previous
import functools

import jax
import jax.numpy as jnp
from jax.experimental import pallas as pl
from jax.experimental.pallas import tpu as pltpu
from jax.experimental.pallas import tpu_sc as plsc

EMBED_DIM = 32
CHUNK = 128
CHUNKS_PER_STEP = 8
IDX_PER_STEP = CHUNK * CHUNKS_PER_STEP
OUT_MINOR = CHUNK * EMBED_DIM
TBLOCK = 8192


def _transpose_table(t_t):
    V = t_t.shape[1]

    def body(in_ref, out_ref):
        s = in_ref[...].T.reshape(TBLOCK // 4, 4, EMBED_DIM)
        out_ref[...] = jnp.concatenate(
            [s[:, k, :] for k in range(4)], axis=1
        )

    return pl.pallas_call(
        body,
        grid=(pl.cdiv(V, TBLOCK),),
        in_specs=[pl.BlockSpec((EMBED_DIM, TBLOCK), lambda i: (0, i))],
        out_specs=pl.BlockSpec((TBLOCK // 4, 128), lambda i: (i, 0)),
        out_shape=jax.ShapeDtypeStruct((V // 4, 128), jnp.float32),
        compiler_params=pltpu.CompilerParams(
            dimension_semantics=("parallel",)
        ),
    )(t_t)


@jax.jit
def kernel(x, table):
    B, L = x.shape
    V = table.shape[0]
    N = B * L
    x_t = jnp.transpose(x)
    table_rm = _transpose_table(jnp.transpose(table)).reshape(V, EMBED_DIM)
    mesh = plsc.VectorSubcoreMesh(core_axis_name="c", subcore_axis_name="s")
    steps_per_l = B // IDX_PER_STEP
    rows_per_step = IDX_PER_STEP * EMBED_DIM // OUT_MINOR

    @functools.partial(
        pl.kernel,
        out_type=jax.ShapeDtypeStruct((N, EMBED_DIM), table.dtype),
        mesh=mesh,
        compiler_params=pltpu.CompilerParams(use_tc_tiling_on_sc=False),
        scratch_types=[pltpu.SemaphoreType.DMA],
    )
    def gather_kernel(table_hbm, idx_hbm, out_hbm, sem):
        def body(idx_vmem, out_vmem):
            copies = [
                pltpu.async_copy(
                    table_hbm.at[idx_vmem.at[0, pl.ds(j * CHUNK, CHUNK)]],
                    out_vmem.at[pl.ds(j * CHUNK, CHUNK)],
                    sem,
                )
                for j in range(CHUNKS_PER_STEP)
            ]
            for c in copies:
                c.wait()

        pltpu.emit_pipeline(
            body,
            grid=(L, steps_per_l),
            in_specs=[
                pl.BlockSpec((1, IDX_PER_STEP), lambda l, i: (l, i))
            ],
            out_specs=[
                pl.BlockSpec(
                    (IDX_PER_STEP, EMBED_DIM),
                    lambda l, i: (l * steps_per_l + i, 0),
                )
            ],
            core_axis_name=("c", "s"),
            dimension_semantics=(pltpu.PARALLEL, pltpu.PARALLEL),
        )(idx_hbm, out_hbm)

    g = gather_kernel(table_rm, x_t)
    return g.reshape(L, B, EMBED_DIM).transpose(1, 0, 2)
